# baseline (speedup 1.0000x reference)
import functools

import jax
import jax.numpy as jnp
from jax import lax
from jax.experimental import pallas as pl
from jax.experimental.pallas import tpu as pltpu
from jax.experimental.pallas import tpu_sc as plsc

_NC, _NS, _L = 2, 16, 16
_NW = _NC * _NS
_NBUF = 2
_TPP = 13
_RPART = 33


def _gather_scale(lidx, wf, kvf, n, p2, U):
    HW, C = kvf.shape[1], kvf.shape[2]
    NPH = n // _NC
    UPB = U // n

    mesh = plsc.VectorSubcoreMesh(core_axis_name="c", subcore_axis_name="s")

    def body(lidx_hbm, wf_hbm, kvf_hbm, out_hbm,
             idx_v, w_v, buf0, buf1, shf,
             g0, g1, s0, s1, stg):
        c = lax.axis_index("c")
        sid = lax.axis_index("s")
        wid = c * _NS + sid
        pltpu.sync_copy(lidx_hbm.at[wid], idx_v)
        pltpu.sync_copy(wf_hbm.at[wid], w_v)
        bufs = (buf0, buf1)
        gsems = (g0, g1)
        ssems = (s0, s1)
        off_s = 12 * sid + jnp.minimum(sid, 4)

        def stage(p, region, rows):
            base = (c * NPH + p) * p2
            per = rows // _NS
            pltpu.async_copy(kvf_hbm.at[pl.ds(base + per * sid, per)],
                             region.at[pl.ds(per * sid, per)], stg)

            @pl.when(sid == 0)
            def _():
                pltpu.async_copy(kvf_hbm.at[pl.ds(base + per * _NS, 1)],
                                 region.at[pl.ds(per * _NS, 1)], stg)

        def stage_wait(p, region, rows):
            per = rows // _NS
            pltpu.make_async_copy(kvf_hbm.at[pl.ds(0, per)],
                                  region.at[pl.ds(0, per)], stg).wait()

            @pl.when(sid == 0)
            def _():
                pltpu.make_async_copy(kvf_hbm.at[pl.ds(0, 1)],
                                      region.at[pl.ds(0, 1)], stg).wait()

        def unit_vec(p, k, ref):
            m = p * _TPP + k
            return ref[m // 8, pl.ds((m % 8) * _L, _L)]

        def gather(p, k, slot, region, rows):
            lval = unit_vec(p, k, idx_v)[0]
            if rows == p2:
                pltpu.async_copy(region.at[pl.ds(lval, 1)], bufs[slot],
                                 gsems[slot])
            else:
                @pl.when(lval < rows)
                def _():
                    pltpu.async_copy(region.at[pl.ds(lval, 1)], bufs[slot],
                                     gsems[slot])

                @pl.when(lval >= rows)
                def _():
                    gval = (c * NPH + p) * p2 + lval
                    pltpu.async_copy(kvf_hbm.at[pl.ds(gval, 1)], bufs[slot],
                                     gsems[slot])

        def gather_wait(slot):
            pltpu.make_async_copy(kvf_hbm.at[pl.ds(0, 1)], bufs[slot],
                                  gsems[slot]).wait()

        def store(p, k, slot):
            orow = (c * NPH + p) * UPB + jnp.minimum(off_s + k, UPB - 1)
            pltpu.async_copy(bufs[slot], out_hbm.at[pl.ds(orow, 1)],
                             ssems[slot])

        def store_wait(slot):
            pltpu.make_async_copy(bufs[slot], out_hbm.at[pl.ds(0, 1)],
                                  ssems[slot]).wait()

        def scale(p, k, slot):
            buf = bufs[slot]
            wvec = unit_vec(p, k, w_v)

            @pl.loop(0, HW)
            def _(h):
                @pl.loop(0, C // _L, unroll=8)
                def _(g):
                    buf[0, h, pl.ds(g * _L, _L)] = (
                        buf[0, h, pl.ds(g * _L, _L)] * wvec)

        def process(p, k, s, wait_store, region, rows):
            gather_wait(s)
            q = k + _NBUF - 1
            sq = (s + _NBUF - 1) % _NBUF
            if wait_store is None:
                @pl.when(q < _TPP)
                def _():
                    store_wait(sq)
                    gather(p, q, sq, region, rows)
            elif wait_store:
                store_wait(sq)
                gather(p, q, sq, region, rows)
            else:
                gather(p, q, sq, region, rows)
            scale(p, k, s)
            store(p, k, s)

        def process_phase(p, region, rows):
            for s in range(_NBUF - 1):
                gather(p, s, s, region, rows)
            process(p, 0, 0, False, region, rows)
            for k in range(1, _NBUF):
                process(p, k, k % _NBUF, True, region, rows)

            @pl.loop(_NBUF, _TPP - 1, step=_NBUF)
            def _(base):
                for s in range(_NBUF):
                    process(p, base + s, s, None, region, rows)

            k = _TPP - 1
            gather_wait(k % _NBUF)
            scale(p, k, k % _NBUF)
            store(p, k, k % _NBUF)
            for s in range(_NBUF):
                store_wait(s)

        for p in range(NPH):
            stage(p, shf, p2)
            stage_wait(p, shf, p2)
            plsc.subcore_barrier()
            process_phase(p, shf, p2)
            if p + 1 < NPH:
                plsc.subcore_barrier()

    f = pl.kernel(
        body,
        out_type=jax.ShapeDtypeStruct((U, HW, C), jnp.float32),
        mesh=mesh,
        scratch_types=[
            pltpu.VMEM((8, 128), jnp.int32),
            pltpu.VMEM((8, 128), jnp.float32),
            pltpu.VMEM((1, HW, C), jnp.float32),
            pltpu.VMEM((1, HW, C), jnp.float32),
            pltpu.VMEM_SHARED((p2, HW, C), jnp.float32),
        ] + [pltpu.SemaphoreType.DMA] * 5,
    )
    return f(lidx, wf, kvf)


def kernel(r_idx, r_weight, kv):
    n, p2, topk = r_idx.shape
    hw, c = kv.shape[2], kv.shape[3]
    U = n * p2 * topk
    assert (n, p2, topk) == (8, 49, 4)
    kvf = kv.reshape(n * p2, hw, c)
    s_ar = jnp.arange(_NS)
    offs = 12 * s_ar + jnp.minimum(s_ar, 4)
    upb = p2 * topk
    inb = jnp.minimum(offs[:, None] + jnp.arange(_TPP)[None, :], upb - 1)
    u = ((jnp.arange(_NC)[:, None, None, None] * (n // _NC)
          + jnp.arange(n // _NC)[None, None, :, None]) * upb
         + inb[None, :, None, :])
    u = u.reshape(_NW, (n // _NC) * _TPP)
    lidx = r_idx.reshape(-1)[u].astype(jnp.int32)
    wsel = r_weight.reshape(-1)[u]
    pad = 8 * 128 - u.shape[1] * _L
    lidx = jnp.pad(jnp.repeat(lidx, _L, axis=1),
                   ((0, 0), (0, pad))).reshape(_NW, 8, 128)
    wf = jnp.pad(jnp.repeat(wsel, _L, axis=1),
                 ((0, 0), (0, pad))).reshape(_NW, 8, 128)
    out = _gather_scale(lidx, wf, kvf, n, p2, U)
    return out.reshape(n, p2, topk, hw, c)


# speedup vs baseline: 1.2515x; 1.2515x over previous
import functools

import jax
import jax.numpy as jnp
from jax import lax
from jax.experimental import pallas as pl
from jax.experimental.pallas import tpu as pltpu
from jax.experimental.pallas import tpu_sc as plsc

_NC, _NS, _L = 2, 16, 16
_NW = _NC * _NS


def _gather_scale(gidx, wf, kvf, upw, *, interpret=False):
    HW, C = kvf.shape[1], kvf.shape[2]
    U = upw * _NW

    mesh = plsc.VectorSubcoreMesh(core_axis_name="c", subcore_axis_name="s")

    NBUF = 4
    assert upw % NBUF == 1 and upw > 2 * NBUF

    def body(gidx_hbm, wf_hbm, kvf_hbm, out_hbm,
             idx_v, w_v, buf0, buf1, buf2, buf3,
             g0, g1, g2, g3, s0, s1, s2, s3):
        wid = lax.axis_index("s") * _NC + lax.axis_index("c")
        pltpu.sync_copy(gidx_hbm.at[wid], idx_v)
        pltpu.sync_copy(wf_hbm.at[wid], w_v)
        bufs = (buf0, buf1, buf2, buf3)
        gsems = (g0, g1, g2, g3)
        ssems = (s0, s1, s2, s3)

        def gather(r, slot):
            gval = idx_v[r // 8, pl.ds((r % 8) * _L, _L)][0]
            row = kvf_hbm.at[pl.ds(gval, 1)]
            h2 = HW // 2
            pltpu.async_copy(row.at[:, pl.ds(0, h2)],
                             bufs[slot].at[:, pl.ds(0, h2)], gsems[slot])
            pltpu.async_copy(row.at[:, pl.ds(h2, h2)],
                             bufs[slot].at[:, pl.ds(h2, h2)], gsems[slot])

        def gather_wait(slot):
            h2 = HW // 2
            for o in (0, h2):
                pltpu.make_async_copy(
                    kvf_hbm.at[pl.ds(0, 1)].at[:, pl.ds(o, h2)],
                    bufs[slot].at[:, pl.ds(o, h2)],
                    gsems[slot]).wait()

        def store(r, slot):
            pltpu.async_copy(bufs[slot], out_hbm.at[pl.ds(wid * upw + r, 1)],
                             ssems[slot])

        def store_wait(slot):
            pltpu.make_async_copy(
                bufs[slot], out_hbm.at[pl.ds(0, 1)], ssems[slot]).wait()

        def scale(r, slot):
            buf = bufs[slot]
            wvec = w_v[r // 8, pl.ds((r % 8) * _L, _L)]

            @pl.loop(0, HW)
            def _(h):
                @pl.loop(0, C // _L, unroll=8)
                def _(g):
                    buf[0, h, pl.ds(g * _L, _L)] = (
                        buf[0, h, pl.ds(g * _L, _L)] * wvec)

        def process(r, s, wait_store):
            gather_wait(s)
            q = r + NBUF - 1
            sq = (s + NBUF - 1) % NBUF
            if wait_store is None:
                @pl.when(q < upw)
                def _():
                    store_wait(sq)
                    gather(q, sq)
            elif wait_store:
                store_wait(sq)
                gather(q, sq)
            else:
                gather(q, sq)
            scale(r, s)
            store(r, s)

        for s in range(NBUF - 1):
            gather(s, s)
        process(0, 0, False)
        for r in range(1, NBUF):
            process(r, r % NBUF, True)

        @pl.loop(NBUF, upw - 1, step=NBUF)
        def _(base):
            for s in range(NBUF):
                process(base + s, s, None)

        gather_wait((upw - 1) % NBUF)
        scale(upw - 1, (upw - 1) % NBUF)
        store(upw - 1, (upw - 1) % NBUF)
        for s in range(NBUF):
            store_wait(s)

    f = pl.kernel(
        body,
        out_type=jax.ShapeDtypeStruct((U, HW, C), jnp.float32),
        mesh=mesh,
        scratch_types=[
            pltpu.VMEM((8, 128), jnp.int32),
            pltpu.VMEM((8, 128), jnp.float32),
            pltpu.VMEM((1, HW, C), jnp.float32),
            pltpu.VMEM((1, HW, C), jnp.float32),
            pltpu.VMEM((1, HW, C), jnp.float32),
            pltpu.VMEM((1, HW, C), jnp.float32),
        ] + [pltpu.SemaphoreType.DMA] * 8,
        interpret=interpret,
    )
    return f(gidx, wf, kvf)


def kernel(r_idx, r_weight, kv):
    n, p2, topk = r_idx.shape
    hw, c = kv.shape[2], kv.shape[3]
    U = n * p2 * topk
    assert U % _NW == 0
    upw = U // _NW
    assert upw <= 128 and upw * _L <= 8 * 128
    kvf = kv.reshape(n * p2, hw, c)
    gflat = (jnp.arange(n, dtype=jnp.int32)[:, None, None] * p2
             + r_idx.astype(jnp.int32)).reshape(_NW, upw)
    grep = jnp.repeat(gflat, _L, axis=1)
    gidx = jnp.pad(grep, ((0, 0), (0, 8 * 128 - upw * _L))).reshape(
        _NW, 8, 128)
    wrep = jnp.repeat(r_weight.reshape(_NW, upw), _L, axis=1)
    wf = jnp.pad(wrep, ((0, 0), (0, 8 * 128 - upw * _L))).reshape(_NW, 8, 128)
    out = _gather_scale(gidx, wf, kvf, upw)
    return out.reshape(n, p2, topk, hw, c)
